# baseline (speedup 1.0000x reference)
import functools

import jax
import jax.numpy as jnp
from jax import lax
from jax.experimental import pallas as pl
from jax.experimental.pallas import tpu as pltpu
from jax.experimental.pallas import tpu_sc as plsc


def _sc_gather(table, idx):
    v, d = table.shape
    b = idx.shape[0]
    nc, ns = 2, 16
    nw = nc * ns
    b_per_w = b // nw
    batch = 16
    mesh = plsc.VectorSubcoreMesh(core_axis_name="c", subcore_axis_name="s")

    @functools.partial(
        pl.kernel,
        mesh=mesh,
        out_type=jax.ShapeDtypeStruct((b, d), table.dtype),
        scratch_types=[
            pltpu.VMEM((b_per_w,), jnp.int32),
            pltpu.VMEM((b_per_w, d), table.dtype),
            pltpu.SemaphoreType.DMA,
        ],
    )
    def k(table_hbm, idx_hbm, out_hbm, idx_v, buf, sem):
        wid = lax.axis_index("s") * nc + lax.axis_index("c")
        base = wid * b_per_w
        pltpu.sync_copy(idx_hbm.at[pl.ds(base, b_per_w)], idx_v)

        @pl.loop(0, b_per_w, step=batch)
        def _(i):
            v = idx_v[pl.ds(i, batch)]
            for j in range(batch):
                tok = v[j]
                pltpu.async_copy(
                    table_hbm.at[pl.ds(tok, 1)], buf.at[pl.ds(i + j, 1)], sem
                )
            for j in range(batch):
                pltpu.make_async_copy(
                    table_hbm.at[pl.ds(0, 1)], buf.at[pl.ds(i + j, 1)], sem
                ).wait()

        pltpu.sync_copy(buf, out_hbm.at[pl.ds(base, b_per_w)])

    return k(table, idx)


def _tc_project(rows, proj_w, scale_arr):
    b, d = rows.shape
    m = proj_w.shape[0]
    mb = 1024
    grid = b // mb

    def body(rows_ref, w_ref, scale_ref, out_ref):
        acc = lax.dot_general(
            rows_ref[...],
            w_ref[...],
            dimension_numbers=(((1,), (1,)), ((), ())),
            preferred_element_type=jnp.float32,
        )
        out_ref[...] = acc * scale_ref[0]

    return pl.pallas_call(
        body,
        grid=(grid,),
        in_specs=[
            pl.BlockSpec((mb, d), lambda i: (i, 0)),
            pl.BlockSpec((m, d), lambda i: (0, 0)),
            pl.BlockSpec(memory_space=pltpu.SMEM),
        ],
        out_specs=pl.BlockSpec((mb, m), lambda i: (i, 0)),
        out_shape=jax.ShapeDtypeStruct((b, m), jnp.float32),
    )(rows, proj_w, scale_arr)


def kernel(token_ids, embed_weight, proj_weight, scale):
    batch, seq = token_ids.shape
    model_dim = proj_weight.shape[0]
    ids = token_ids.reshape(-1).astype(jnp.int32)
    rows = _sc_gather(embed_weight, ids)
    scale_arr = jnp.reshape(scale, (1,)).astype(jnp.float32)
    out = _tc_project(rows, proj_weight, scale_arr)
    return out.reshape(batch, seq, model_dim)


# speedup vs baseline: 1.2079x; 1.2079x over previous
import functools

import jax
import jax.numpy as jnp
from jax import lax
from jax.experimental import pallas as pl
from jax.experimental.pallas import tpu as pltpu
from jax.experimental.pallas import tpu_sc as plsc


def _tc_pair_pack(table_t, half):
    d, v = table_t.shape
    n_lane_blocks = half // 128
    k = 17
    steps = n_lane_blocks // k
    blk = 128 * k

    def body(left_ref, right_ref, out_ref):
        left = jnp.transpose(left_ref[...])
        right = jnp.transpose(right_ref[...])
        out_ref[...] = jnp.concatenate([left, right], axis=1)

    return pl.pallas_call(
        body,
        grid=(steps,),
        in_specs=[
            pl.BlockSpec((d, blk), lambda j: (0, j)),
            pl.BlockSpec((d, blk), lambda j: (0, steps + j)),
        ],
        out_specs=pl.BlockSpec((blk, 2 * d), lambda j: (j, 0)),
        out_shape=jax.ShapeDtypeStruct((half, 2 * d), jnp.float32),
    )(table_t, table_t)


def _sc_gather(table, idx):
    v, d = table.shape
    b = idx.shape[0]
    nc, ns = 2, 16
    nw = nc * ns
    b_per_w = b // nw
    ch = 128
    n_chunks = b_per_w // ch
    mesh = plsc.VectorSubcoreMesh(core_axis_name="c", subcore_axis_name="s")

    @functools.partial(
        pl.kernel,
        mesh=mesh,
        out_type=jax.ShapeDtypeStruct((b, d), table.dtype),
        scratch_types=[
            pltpu.VMEM((b_per_w,), jnp.int32),
            pltpu.VMEM((ch, d), table.dtype),
            pltpu.SemaphoreType.DMA,
        ],
    )
    def k(table_hbm, idx_hbm, out_hbm, idx_v, buf, sem):
        wid = lax.axis_index("s") * nc + lax.axis_index("c")
        base = wid * b_per_w
        pltpu.sync_copy(idx_hbm.at[pl.ds(base, b_per_w)], idx_v)
        for j in range(n_chunks):
            pltpu.async_copy(
                table_hbm.at[idx_v.at[pl.ds(j * ch, ch)]], buf, sem
            ).wait()
            pltpu.sync_copy(buf, out_hbm.at[pl.ds(base + j * ch, ch)])

    return k(table, idx)


def _tc_project(rows2, ids3, proj_w, scale_arr, half):
    b = rows2.shape[0]
    m, d = proj_w.shape
    mb = 1024
    grid = b // mb

    def body(rows_ref, ids_ref, w_ref, scale_ref, out_ref):
        sel = (ids_ref[0, 0, :] >= half).astype(jnp.int32)
        sel = jnp.reshape(sel, (mb, 1))
        rows = rows_ref[...]
        h = jnp.where(sel == 1, rows[:, d:], rows[:, :d])
        acc = lax.dot_general(
            h,
            w_ref[...],
            dimension_numbers=(((1,), (1,)), ((), ())),
            preferred_element_type=jnp.float32,
        )
        out_ref[...] = acc * scale_ref[0]

    return pl.pallas_call(
        body,
        grid=(grid,),
        in_specs=[
            pl.BlockSpec((mb, 2 * d), lambda i: (i, 0)),
            pl.BlockSpec((1, 1, mb), lambda i: (i, 0, 0)),
            pl.BlockSpec((m, d), lambda i: (0, 0)),
            pl.BlockSpec(memory_space=pltpu.SMEM),
        ],
        out_specs=pl.BlockSpec((mb, m), lambda i: (i, 0)),
        out_shape=jax.ShapeDtypeStruct((b, m), jnp.float32),
    )(rows2, ids3, proj_w, scale_arr)


def kernel(token_ids, embed_weight, proj_weight, scale):
    batch, seq = token_ids.shape
    v, d = embed_weight.shape
    half = 50048
    model_dim = proj_weight.shape[0]
    ids = token_ids.reshape(-1).astype(jnp.int32)
    table_t = jnp.swapaxes(embed_weight, 0, 1)
    pairs = _tc_pair_pack(table_t, half)
    idx = jnp.where(ids >= half, ids - half, ids)
    rows2 = _sc_gather(pairs, idx)
    ids3 = ids.reshape(-1, 1, 1024)
    scale_arr = jnp.reshape(scale, (1,)).astype(jnp.float32)
    out = _tc_project(rows2, ids3, proj_weight, scale_arr, half)
    return out.reshape(batch, seq, model_dim)


# speedup vs baseline: 1.2597x; 1.0429x over previous
import functools

import jax
import jax.numpy as jnp
from jax import lax
from jax.experimental import pallas as pl
from jax.experimental.pallas import tpu as pltpu
from jax.experimental.pallas import tpu_sc as plsc


def _tc_pair_pack(table_t, half):
    d, v = table_t.shape
    n_lane_blocks = half // 128
    k = 17
    steps = n_lane_blocks // k
    blk = 128 * k

    def body(left_ref, right_ref, out_ref):
        left = jnp.transpose(left_ref[...])
        right = jnp.transpose(right_ref[...])
        out_ref[...] = jnp.concatenate([left, right], axis=1)

    return pl.pallas_call(
        body,
        grid=(steps,),
        in_specs=[
            pl.BlockSpec((d, blk), lambda j: (0, j)),
            pl.BlockSpec((d, blk), lambda j: (0, steps + j)),
        ],
        out_specs=pl.BlockSpec((blk, 2 * d), lambda j: (j, 0)),
        out_shape=jax.ShapeDtypeStruct((half, 2 * d), jnp.float32),
    )(table_t, table_t)


def _sc_gather(table, idx):
    v, d = table.shape
    b = idx.shape[0]
    nc, ns = 2, 16
    nw = nc * ns
    b_per_w = b // nw
    ch = 128
    n_chunks = b_per_w // ch
    mesh = plsc.VectorSubcoreMesh(core_axis_name="c", subcore_axis_name="s")

    @functools.partial(
        pl.kernel,
        mesh=mesh,
        out_type=jax.ShapeDtypeStruct((b, d), table.dtype),
        scratch_types=[
            pltpu.VMEM((b_per_w,), jnp.int32),
            pltpu.VMEM((ch, d), table.dtype),
            pltpu.SemaphoreType.DMA,
        ],
    )
    def k(table_hbm, idx_hbm, out_hbm, idx_v, buf, sem):
        wid = lax.axis_index("s") * nc + lax.axis_index("c")
        base = wid * b_per_w
        pltpu.sync_copy(idx_hbm.at[pl.ds(base, b_per_w)], idx_v)
        for j in range(n_chunks):
            pltpu.async_copy(
                table_hbm.at[idx_v.at[pl.ds(j * ch, ch)]], buf, sem
            ).wait()
            pltpu.sync_copy(buf, out_hbm.at[pl.ds(base + j * ch, ch)])

    return k(table, idx)


def _tc_project(rows2, ids3, proj_w, scale_arr, half):
    b = rows2.shape[0]
    m, d = proj_w.shape
    mb = 2048
    grid = b // mb

    def body(rows_ref, ids_ref, w_ref, scale_ref, out_ref):
        sel = (ids_ref[0, 0, :] >= half).astype(jnp.int32)
        sel = jnp.reshape(sel, (mb, 1))
        rows = rows_ref[...]
        h = jnp.where(sel == 1, rows[:, d:], rows[:, :d]).astype(jnp.bfloat16)
        acc = lax.dot_general(
            h,
            w_ref[...].astype(jnp.bfloat16),
            dimension_numbers=(((1,), (1,)), ((), ())),
            preferred_element_type=jnp.float32,
        )
        out_ref[...] = acc * scale_ref[0]

    return pl.pallas_call(
        body,
        grid=(grid,),
        in_specs=[
            pl.BlockSpec((mb, 2 * d), lambda i: (i, 0)),
            pl.BlockSpec((1, 1, mb), lambda i: (i, 0, 0)),
            pl.BlockSpec((m, d), lambda i: (0, 0)),
            pl.BlockSpec(memory_space=pltpu.SMEM),
        ],
        out_specs=pl.BlockSpec((mb, m), lambda i: (i, 0)),
        out_shape=jax.ShapeDtypeStruct((b, m), jnp.float32),
    )(rows2, ids3, proj_w, scale_arr)


def kernel(token_ids, embed_weight, proj_weight, scale):
    batch, seq = token_ids.shape
    v, d = embed_weight.shape
    half = 50048
    model_dim = proj_weight.shape[0]
    ids = token_ids.reshape(-1).astype(jnp.int32)
    table_t = jnp.swapaxes(embed_weight, 0, 1)
    pairs = _tc_pair_pack(table_t, half)
    idx = jnp.where(ids >= half, ids - half, ids)
    rows2 = _sc_gather(pairs, idx)
    ids3 = ids.reshape(-1, 1, 2048)
    scale_arr = jnp.reshape(scale, (1,)).astype(jnp.float32)
    out = _tc_project(rows2, ids3, proj_weight, scale_arr, half)
    return out.reshape(batch, seq, model_dim)
